# baseline (speedup 1.0000x reference)
import functools

import jax
import jax.numpy as jnp
from jax import lax
from jax.experimental import pallas as pl
from jax.experimental.pallas import tpu as pltpu
from jax.experimental.pallas import tpu_sc as plsc

BATCH = 4096
SEQ = 200
EMB = 64
LANES = 16
NUM_WORKERS = 32
B_PER_W = BATCH // NUM_WORKERS
NBUF = 2
NEG_INF = float("-inf")


def _fire_gather(table_hbm, idx_all, hbuf, buf, sem, c):
    for i in range(SEQ // LANES + 1):
        o = min(LANES * i, SEQ - LANES)
        hbuf[pl.ds(o, LANES)] = lax.shift_right_logical(
            idx_all[pl.ds(SEQ * c + o, LANES)], 1
        )
    pltpu.async_copy(table_hbm.at[hbuf], buf, sem)


def _wait_gather(table_hbm, buf, sem):
    pltpu.make_async_copy(table_hbm.at[pl.ds(0, SEQ)], buf, sem).wait()


def _reduce_row(buf, idx_all, c, out_v, dst):
    nchunk = EMB // LANES
    init = tuple(jnp.full((LANES,), NEG_INF, jnp.float32) for _ in range(nchunk))

    def body(r, accs):
        idx = plsc.load_gather(
            idx_all, [jnp.full((LANES,), SEQ * c, jnp.int32) + r]
        )
        hi = (idx & 1) == 1
        out = []
        for k in range(nchunk):
            lo_v = buf[r, pl.ds(LANES * k, LANES)]
            hi_v = buf[r, pl.ds(EMB + LANES * k, LANES)]
            out.append(jnp.maximum(accs[k], jnp.where(hi, hi_v, lo_v)))
        return tuple(out)

    accs = lax.fori_loop(0, SEQ, body, init, unroll=4)
    for k in range(nchunk):
        out_v[dst, pl.ds(LANES * k, LANES)] = accs[k]


def _bow_encode(ids, table2):
    mesh = plsc.VectorSubcoreMesh(core_axis_name="c", subcore_axis_name="s")

    @functools.partial(
        pl.kernel,
        out_type=jax.ShapeDtypeStruct((BATCH, EMB), jnp.float32),
        mesh=mesh,
        scratch_types=[
            pltpu.VMEM((B_PER_W * SEQ,), jnp.int32),
            pltpu.VMEM((SEQ,), jnp.int32),
            pltpu.VMEM((SEQ,), jnp.int32),
            pltpu.VMEM((SEQ, 2 * EMB), jnp.float32),
            pltpu.VMEM((SEQ, 2 * EMB), jnp.float32),
            pltpu.VMEM((B_PER_W, EMB), jnp.float32),
            pltpu.SemaphoreType.DMA,
            pltpu.SemaphoreType.DMA,
        ],
        compiler_params=pltpu.CompilerParams(
            use_tc_tiling_on_sc=True, needs_layout_passes=False
        ),
    )
    def k(ids_hbm, table_hbm, out_hbm, idx_all, h0, h1, b0, b1, out_v, s0, s1):
        wid = lax.axis_index("s") * 2 + lax.axis_index("c")
        base = wid * B_PER_W
        hbufs = (h0, h1)
        bufs = (b0, b1)
        sems = (s0, s1)

        pltpu.sync_copy(ids_hbm.at[pl.ds(base * SEQ, B_PER_W * SEQ)], idx_all)
        _fire_gather(table_hbm, idx_all, hbufs[0], bufs[0], sems[0], 0)

        def body(j, _):
            c = j * NBUF
            for b in range(NBUF):
                bn = (b + NBUF - 1) % NBUF

                @pl.when(c + b + NBUF - 1 < B_PER_W)
                def _():
                    _fire_gather(
                        table_hbm, idx_all, hbufs[bn], bufs[bn], sems[bn],
                        c + b + NBUF - 1,
                    )

                _wait_gather(table_hbm, bufs[b], sems[b])
                _reduce_row(bufs[b], idx_all, c + b, out_v, c + b)
            return 0

        lax.fori_loop(0, B_PER_W // NBUF, body, 0)
        pltpu.sync_copy(out_v, out_hbm.at[pl.ds(base, B_PER_W)])

    return k(ids, table2)


def kernel(input, emb_weight):
    ids = jnp.asarray(input, jnp.int32).reshape(-1)
    table2 = emb_weight.reshape(-1, 2 * EMB)
    return _bow_encode(ids, table2)


# speedup vs baseline: 1.1699x; 1.1699x over previous
import functools

import jax
import jax.numpy as jnp
from jax import lax
from jax.experimental import pallas as pl
from jax.experimental.pallas import tpu as pltpu
from jax.experimental.pallas import tpu_sc as plsc

BATCH = 4096
SEQ = 200
EMB = 64
LANES = 16
NUM_WORKERS = 32
B_PER_W = BATCH // NUM_WORKERS
ROWS_PER_CHUNK = 2
CHUNK = ROWS_PER_CHUNK * SEQ
N_CHUNKS = B_PER_W // ROWS_PER_CHUNK
NBUF = 3


def _fire_gather(table_hbm, idx_all, buf, sem, c):
    pltpu.async_copy(table_hbm.at[idx_all.at[c]], buf, sem)


def _wait_gather(table_hbm, buf, sem):
    pltpu.make_async_copy(table_hbm.at[pl.ds(0, CHUNK)], buf, sem).wait()


def _reduce_row(buf, r0, out_v, dst):
    accs = tuple(buf[r0, pl.ds(LANES * c, LANES)] for c in range(EMB // LANES))

    def body(r, accs):
        return tuple(
            jnp.maximum(accs[c], buf[r, pl.ds(LANES * c, LANES)])
            for c in range(EMB // LANES)
        )

    accs = lax.fori_loop(r0 + 1, r0 + SEQ, body, accs, unroll=8)
    for c in range(EMB // LANES):
        out_v[dst, pl.ds(LANES * c, LANES)] = accs[c]


def _bow_encode(ids2, table):
    mesh = plsc.VectorSubcoreMesh(core_axis_name="c", subcore_axis_name="s")

    @functools.partial(
        pl.kernel,
        out_type=jax.ShapeDtypeStruct((BATCH, EMB), jnp.float32),
        mesh=mesh,
        scratch_types=[
            pltpu.VMEM((N_CHUNKS, CHUNK), jnp.int32),
            pltpu.VMEM((CHUNK, EMB), jnp.float32),
            pltpu.VMEM((CHUNK, EMB), jnp.float32),
            pltpu.VMEM((CHUNK, EMB), jnp.float32),
            pltpu.VMEM((B_PER_W, EMB), jnp.float32),
            pltpu.SemaphoreType.DMA,
            pltpu.SemaphoreType.DMA,
            pltpu.SemaphoreType.DMA,
        ],
        compiler_params=pltpu.CompilerParams(use_tc_tiling_on_sc=False),
    )
    def k(ids_hbm, table_hbm, out_hbm, idx_all, b0, b1, b2, out_v, s0, s1, s2):
        wid = lax.axis_index("s") * 2 + lax.axis_index("c")
        base = wid * B_PER_W
        bufs = (b0, b1, b2)
        sems = (s0, s1, s2)

        pltpu.sync_copy(ids_hbm.at[pl.ds(N_CHUNKS * wid, N_CHUNKS)], idx_all)
        for c in range(NBUF - 1):
            _fire_gather(table_hbm, idx_all, bufs[c], sems[c], c)

        def process(chunk, b):
            _wait_gather(table_hbm, bufs[b], sems[b])
            for r in range(ROWS_PER_CHUNK):
                _reduce_row(bufs[b], r * SEQ, out_v, chunk * ROWS_PER_CHUNK + r)

        def body(j, _):
            c = j * NBUF
            for b in range(NBUF):
                bn = (b + NBUF - 1) % NBUF

                @pl.when(c + b + NBUF - 1 < N_CHUNKS)
                def _():
                    _fire_gather(
                        table_hbm, idx_all, bufs[bn], sems[bn], c + b + NBUF - 1
                    )

                process(c + b, b)
            return 0

        lax.fori_loop(0, N_CHUNKS // NBUF, body, 0)
        for c in range(N_CHUNKS - N_CHUNKS % NBUF, N_CHUNKS):
            process(c, c % NBUF)
        pltpu.sync_copy(out_v, out_hbm.at[pl.ds(base, B_PER_W)])

    return k(ids2, table)


def kernel(input, emb_weight):
    ids2 = jnp.asarray(input, jnp.int32).reshape(BATCH // ROWS_PER_CHUNK, CHUNK)
    return _bow_encode(ids2, emb_weight)
